# baseline (speedup 1.0000x reference)
import functools

import jax
import jax.numpy as jnp
from jax import lax
from jax.experimental import pallas as pl
from jax.experimental.pallas import tpu as pltpu
from jax.experimental.pallas import tpu_sc as plsc

_N = 10000
_E = 320000
_D = 128
_G = 128
_C = 10

_NC = 2
_NS = 16
_NW = _NC * _NS
_CH = 128
_NCHP = 80
_EPWP = _CH * _NCHP
_EPW = _E // _NW
_NP = _N + 8
_NPAD = _NW * (_EPWP - _EPW)
_RB = 632
_GPS = _G // _NS
_ZB = 80
_NZB = _N // _ZB

_mesh_args = dict(core_axis_name="c", subcore_axis_name="s")


@functools.cache
def _sc_kernels():
    mesh = plsc.VectorSubcoreMesh(**_mesh_args)

    deg = functools.partial(
        pl.kernel,
        out_type=jax.ShapeDtypeStruct((_NC, _N, _D), jnp.float32),
        mesh=mesh,
        scratch_types=[
            pltpu.VMEM((_NCHP, _CH), jnp.int32),
            pltpu.VMEM((_CH, _D), jnp.float32),
            pltpu.VMEM_SHARED((_N, _D), jnp.float32),
        ],
    )(_deg_body)
    prop = functools.partial(
        pl.kernel,
        out_type=jax.ShapeDtypeStruct((_NC, _N, _D), jnp.float32),
        mesh=mesh,
        scratch_types=[
            pltpu.VMEM((_NCHP // 2, _CH), jnp.int32),
            pltpu.VMEM((_NCHP // 2, _CH), jnp.int32),
            pltpu.VMEM((_CH, _D), jnp.float32),
            pltpu.VMEM((_CH, _D), jnp.float32),
            pltpu.VMEM_SHARED((_N, _D), jnp.float32),
            pltpu.SemaphoreType.DMA,
            pltpu.SemaphoreType.DMA,
            pltpu.SemaphoreType.DMA,
            pltpu.SemaphoreType.DMA,
        ],
    )(_prop_body)
    pool = functools.partial(
        pl.kernel,
        out_type=jax.ShapeDtypeStruct((_NC, _G, _D), jnp.float32),
        mesh=mesh,
        scratch_types=[
            pltpu.VMEM((_ZB,), jnp.int32),
            pltpu.VMEM((_ZB, _D), jnp.float32),
            pltpu.VMEM_SHARED((_G, _D), jnp.float32),
        ],
    )(_pool_body)
    return deg, prop, pool


def _zero_acc(s, zeros_hbm, acc):
    @pl.when(s < _NS - 1)
    def _():
        off = pl.multiple_of(s * _RB, 8)
        pltpu.sync_copy(zeros_hbm.at[pl.ds(off, _RB)], acc.at[pl.ds(off, _RB)])

    @pl.when(s == _NS - 1)
    def _():
        base = (_NS - 1) * _RB
        pltpu.sync_copy(zeros_hbm.at[pl.ds(base, _N - base)],
                        acc.at[pl.ds(base, _N - base)])


def _writeback(c, s, acc, out_hbm):
    @pl.loop(0, _NZB // _NS + 1)
    def _(j):
        bid = s + j * _NS

        @pl.when(bid < _NZB)
        def _():
            off = pl.multiple_of(bid * _ZB, 8)
            pltpu.sync_copy(acc.at[pl.ds(off, _ZB)],
                            out_hbm.at[c, pl.ds(off, _ZB)])


def _deg_body(dst_hbm, ones_hbm, zeros_hbm, out_hbm, didx, ones_v, acc):
    c = lax.axis_index("c")
    s = lax.axis_index("s")
    wid = c * _NS + s
    _zero_acc(s, zeros_hbm, acc)
    pltpu.sync_copy(dst_hbm.at[wid], didx)
    pltpu.sync_copy(ones_hbm, ones_v)
    plsc.subcore_barrier()

    @pl.loop(0, _NCHP)
    def _(i):
        pltpu.sync_copy(ones_v, acc.at[didx.at[i]], add=True)

    plsc.subcore_barrier()
    _writeback(c, s, acc, out_hbm)


def _prop_body(t_hbm, src_hbm, dst_hbm, zeros_hbm, out_hbm,
               sidx, didx, rows0, rows1, acc, gsem0, gsem1, ssem0, ssem1):
    c = lax.axis_index("c")
    s = lax.axis_index("s")
    wid = c * _NS + s
    _zero_acc(s, zeros_hbm, acc)
    plsc.subcore_barrier()
    half_n = _NCHP // 2

    @pl.loop(0, 2)
    def _(half):
        hb = pl.multiple_of(half * half_n, 8)
        pltpu.sync_copy(src_hbm.at[wid, pl.ds(hb, half_n)], sidx)
        pltpu.sync_copy(dst_hbm.at[wid, pl.ds(hb, half_n)], didx)

        @pl.loop(0, half_n)
        def _(i):
            pltpu.async_copy(t_hbm.at[sidx.at[i]], rows0, gsem0).wait()
            pltpu.sync_copy(rows0, acc.at[didx.at[i]], add=True)

    plsc.subcore_barrier()
    _writeback(c, s, acc, out_hbm)


def _pool_body(h_hbm, batch_hbm, zeros_hbm, out_hbm, idx_v, rows, acc):
    c = lax.axis_index("c")
    s = lax.axis_index("s")
    wid = c * _NS + s
    pltpu.sync_copy(zeros_hbm.at[pl.ds(s * _GPS, _GPS)],
                    acc.at[pl.ds(s * _GPS, _GPS)])
    plsc.subcore_barrier()

    @pl.loop(0, _NZB // _NW + 1)
    def _(j):
        cid = wid + j * _NW

        @pl.when(cid < _NZB)
        def _():
            off = pl.multiple_of(cid * _ZB, 8)
            pltpu.sync_copy(batch_hbm.at[pl.ds(off, _ZB)], idx_v)
            pltpu.sync_copy(h_hbm.at[pl.ds(off, _ZB)], rows)
            pltpu.sync_copy(rows, acc.at[idx_v], add=True)

    plsc.subcore_barrier()
    pltpu.sync_copy(acc.at[pl.ds(s * _GPS, _GPS)],
                    out_hbm.at[c, pl.ds(s * _GPS, _GPS)])


def _dot(a, b):
    return jnp.dot(a, b, preferred_element_type=jnp.float32,
                   precision=lax.Precision.DEFAULT)


def _first_body(x_ref, w_ref, o_ref):
    o_ref[pl.ds(0, _N), :] = _dot(x_ref[...], w_ref[...])
    o_ref[pl.ds(_N, 8), :] = jnp.zeros((8, _D), jnp.float32)


_first_call = pl.pallas_call(
    _first_body, out_shape=jax.ShapeDtypeStruct((_NP, _D), jnp.float32))


def _dinv_scale_body(degacc_ref, t_ref, dinv_ref, o_ref):
    deg = degacc_ref[0][:, 0:1] + degacc_ref[1][:, 0:1] + 1.0
    row = lax.broadcasted_iota(jnp.int32, (_N, 1), 0)
    deg = deg - jnp.where(row == 0, float(_NPAD), 0.0)
    dinv = lax.rsqrt(jnp.maximum(deg, 1.0))
    dinv_ref[...] = dinv
    o_ref[pl.ds(0, _N), :] = t_ref[pl.ds(0, _N), :] * dinv
    o_ref[pl.ds(_N, 8), :] = jnp.zeros((8, _D), jnp.float32)


_dinv_scale_call = pl.pallas_call(
    _dinv_scale_body,
    out_shape=(jax.ShapeDtypeStruct((_N, 1), jnp.float32),
               jax.ShapeDtypeStruct((_NP, _D), jnp.float32)))


def _postbn(acc_ref, t_ref, dinv_ref, b_ref, g_ref, be_ref):
    t = t_ref[pl.ds(0, _N), :]
    u = (acc_ref[0] + acc_ref[1] + t) * dinv_ref[...] + b_ref[...]
    mean = jnp.mean(u, axis=0, keepdims=True)
    var = jnp.mean((u - mean) ** 2, axis=0, keepdims=True)
    return jnp.maximum(
        (u - mean) * lax.rsqrt(var + 1e-5) * g_ref[...] + be_ref[...], 0.0)


def _mid_body(acc_ref, t_ref, dinv_ref, b_ref, g_ref, be_ref, w_ref, o_ref):
    h = _postbn(acc_ref, t_ref, dinv_ref, b_ref, g_ref, be_ref)
    o_ref[pl.ds(0, _N), :] = _dot(h, w_ref[...]) * dinv_ref[...]
    o_ref[pl.ds(_N, 8), :] = jnp.zeros((8, _D), jnp.float32)


_mid_call = pl.pallas_call(
    _mid_body, out_shape=jax.ShapeDtypeStruct((_NP, _D), jnp.float32))


def _last_body(acc_ref, t_ref, dinv_ref, b_ref, g_ref, be_ref, o_ref):
    o_ref[...] = _postbn(acc_ref, t_ref, dinv_ref, b_ref, g_ref, be_ref)


_last_call = pl.pallas_call(
    _last_body, out_shape=jax.ShapeDtypeStruct((_N, _D), jnp.float32))


def _cls_body(p_ref, w_ref, b_ref, o_ref):
    o_ref[...] = _dot(p_ref[0] + p_ref[1], w_ref[...]) + b_ref[...]


_cls_call = pl.pallas_call(
    _cls_body, out_shape=jax.ShapeDtypeStruct((_G, _C), jnp.float32))


def _pad_edges(e, fill):
    e2 = e.reshape(_NW, _EPW)
    e2 = jnp.pad(e2, ((0, 0), (0, _EPWP - _EPW)), constant_values=fill)
    return e2.reshape(_NW, _NCHP, _CH)


def kernel(x, edge_index, batch, W0, b0, gamma0, beta0, W1, b1, gamma1,
           beta1, W2, b2, gamma2, beta2, cW, cb):
    src = _pad_edges(edge_index[0], _N)
    dst = _pad_edges(edge_index[1], 0)
    zeros_np = jnp.zeros((_N, _D), jnp.float32)
    ones_ch = jnp.ones((_CH, _D), jnp.float32)

    deg_kernel, prop_kernel, pool_kernel = _sc_kernels()

    degacc = deg_kernel(dst, ones_ch, zeros_np)
    xw0 = _first_call(x, W0)
    dinv, t = _dinv_scale_call(degacc, xw0)

    for (b, g, be, Wn) in ((b0, gamma0, beta0, W1), (b1, gamma1, beta1, W2)):
        acc = prop_kernel(t, src, dst, zeros_np)
        t = _mid_call(acc, t, dinv, b.reshape(1, _D), g.reshape(1, _D),
                      be.reshape(1, _D), Wn)
    acc = prop_kernel(t, src, dst, zeros_np)
    h = _last_call(acc, t, dinv, b2.reshape(1, _D), gamma2.reshape(1, _D),
                   beta2.reshape(1, _D))
    pacc = pool_kernel(h, batch, zeros_np)
    return _cls_call(pacc, cW, cb.reshape(1, _C))


# speedup vs baseline: 1.4098x; 1.4098x over previous
import functools

import jax
import jax.numpy as jnp
from jax import lax
from jax.experimental import pallas as pl
from jax.experimental.pallas import tpu as pltpu
from jax.experimental.pallas import tpu_sc as plsc

_N = 10000
_E = 320000
_D = 128
_G = 128
_C = 10

_NC = 2
_NS = 16
_NW = _NC * _NS
_CH = 128
_NCHP = 80
_EPWP = _CH * _NCHP
_EPW = _E // _NW
_NP = _N + 8
_NPAD = _NW * (_EPWP - _EPW)
_RB = 632
_GPS = _G // _NS
_PCH = 80
_PNCH = _EPW // _PCH
_ZB = 80
_NZB = _N // _ZB

_mesh_args = dict(core_axis_name="c", subcore_axis_name="s")


@functools.cache
def _sc_kernels():
    mesh = plsc.VectorSubcoreMesh(**_mesh_args)

    deg = functools.partial(
        pl.kernel,
        out_type=jax.ShapeDtypeStruct((_NC, _N, _D), jnp.float32),
        mesh=mesh,
        scratch_types=[
            pltpu.VMEM((_NCHP, _CH), jnp.int32),
            pltpu.VMEM((_CH, _D), jnp.float32),
            pltpu.VMEM_SHARED((_N, _D), jnp.float32),
        ],
    )(_deg_body)
    prop = functools.partial(
        pl.kernel,
        out_type=jax.ShapeDtypeStruct((_NC, _N, _D), jnp.float32),
        mesh=mesh,
        scratch_types=[
            pltpu.VMEM((_PCH,), jnp.int32),
            pltpu.VMEM((_PCH,), jnp.int32),
            pltpu.VMEM((_PCH, _D), jnp.float32),
            pltpu.VMEM_SHARED((_N, _D), jnp.float32),
            pltpu.SemaphoreType.DMA,
        ],
    )(_prop_body)
    pool = functools.partial(
        pl.kernel,
        out_type=jax.ShapeDtypeStruct((_NC, _G, _D), jnp.float32),
        mesh=mesh,
        scratch_types=[
            pltpu.VMEM((_ZB,), jnp.int32),
            pltpu.VMEM((_ZB, _D), jnp.float32),
            pltpu.VMEM_SHARED((_G, _D), jnp.float32),
        ],
    )(_pool_body)
    return deg, prop, pool


def _zero_acc(s, zeros_hbm, acc):
    @pl.when(s < _NS - 1)
    def _():
        off = pl.multiple_of(s * _RB, 8)
        pltpu.sync_copy(zeros_hbm.at[pl.ds(off, _RB)], acc.at[pl.ds(off, _RB)])

    @pl.when(s == _NS - 1)
    def _():
        base = (_NS - 1) * _RB
        pltpu.sync_copy(zeros_hbm.at[pl.ds(base, _N - base)],
                        acc.at[pl.ds(base, _N - base)])


def _writeback(c, s, acc, out_hbm):
    @pl.loop(0, _NZB // _NS + 1)
    def _(j):
        bid = s + j * _NS

        @pl.when(bid < _NZB)
        def _():
            off = pl.multiple_of(bid * _ZB, 8)
            pltpu.sync_copy(acc.at[pl.ds(off, _ZB)],
                            out_hbm.at[c, pl.ds(off, _ZB)])


def _deg_body(dst_hbm, ones_hbm, zeros_hbm, out_hbm, didx, ones_v, acc):
    c = lax.axis_index("c")
    s = lax.axis_index("s")
    wid = c * _NS + s
    _zero_acc(s, zeros_hbm, acc)
    pltpu.sync_copy(dst_hbm.at[wid], didx)
    pltpu.sync_copy(ones_hbm, ones_v)
    plsc.subcore_barrier()

    @pl.loop(0, _NCHP)
    def _(i):
        pltpu.sync_copy(ones_v, acc.at[didx.at[i]], add=True)

    plsc.subcore_barrier()
    _writeback(c, s, acc, out_hbm)


def _prop_body(t_hbm, src_hbm, dst_hbm, zeros_hbm, out_hbm,
               sidx, didx, rows, acc, sem):
    c = lax.axis_index("c")
    s = lax.axis_index("s")
    wid = c * _NS + s
    _zero_acc(s, zeros_hbm, acc)
    plsc.subcore_barrier()
    base = wid * _EPW

    @pl.loop(0, _PNCH)
    def _(i):
        off = pl.multiple_of(base + i * _PCH, 8)
        pltpu.sync_copy(src_hbm.at[pl.ds(off, _PCH)], sidx)
        pltpu.sync_copy(dst_hbm.at[pl.ds(off, _PCH)], didx)
        pltpu.async_copy(t_hbm.at[sidx], rows, sem).wait()
        pltpu.sync_copy(rows, acc.at[didx], add=True)

    plsc.subcore_barrier()
    _writeback(c, s, acc, out_hbm)


def _pool_body(h_hbm, batch_hbm, zeros_hbm, out_hbm, idx_v, rows, acc):
    c = lax.axis_index("c")
    s = lax.axis_index("s")
    wid = c * _NS + s
    pltpu.sync_copy(zeros_hbm.at[pl.ds(s * _GPS, _GPS)],
                    acc.at[pl.ds(s * _GPS, _GPS)])
    plsc.subcore_barrier()

    @pl.loop(0, _NZB // _NW + 1)
    def _(j):
        cid = wid + j * _NW

        @pl.when(cid < _NZB)
        def _():
            off = pl.multiple_of(cid * _ZB, 8)
            pltpu.sync_copy(batch_hbm.at[pl.ds(off, _ZB)], idx_v)
            pltpu.sync_copy(h_hbm.at[pl.ds(off, _ZB)], rows)
            pltpu.sync_copy(rows, acc.at[idx_v], add=True)

    plsc.subcore_barrier()
    pltpu.sync_copy(acc.at[pl.ds(s * _GPS, _GPS)],
                    out_hbm.at[c, pl.ds(s * _GPS, _GPS)])


def _dot(a, b):
    return jnp.dot(a, b, preferred_element_type=jnp.float32,
                   precision=lax.Precision.DEFAULT)


def _first_body(x_ref, w_ref, o_ref):
    o_ref[pl.ds(0, _N), :] = _dot(x_ref[...], w_ref[...])
    o_ref[pl.ds(_N, 8), :] = jnp.zeros((8, _D), jnp.float32)


_first_call = pl.pallas_call(
    _first_body, out_shape=jax.ShapeDtypeStruct((_NP, _D), jnp.float32))


def _dinv_scale_body(degacc_ref, t_ref, dinv_ref, o_ref):
    deg = degacc_ref[0][:, 0:1] + degacc_ref[1][:, 0:1] + 1.0
    row = lax.broadcasted_iota(jnp.int32, (_N, 1), 0)
    deg = deg - jnp.where(row == 0, float(_NPAD), 0.0)
    dinv = lax.rsqrt(jnp.maximum(deg, 1.0))
    dinv_ref[...] = dinv
    o_ref[pl.ds(0, _N), :] = t_ref[pl.ds(0, _N), :] * dinv
    o_ref[pl.ds(_N, 8), :] = jnp.zeros((8, _D), jnp.float32)


_dinv_scale_call = pl.pallas_call(
    _dinv_scale_body,
    out_shape=(jax.ShapeDtypeStruct((_N, 1), jnp.float32),
               jax.ShapeDtypeStruct((_NP, _D), jnp.float32)))


def _postbn(acc_ref, t_ref, dinv_ref, b_ref, g_ref, be_ref):
    t = t_ref[pl.ds(0, _N), :]
    u = (acc_ref[0] + acc_ref[1] + t) * dinv_ref[...] + b_ref[...]
    mean = jnp.mean(u, axis=0, keepdims=True)
    var = jnp.mean((u - mean) ** 2, axis=0, keepdims=True)
    return jnp.maximum(
        (u - mean) * lax.rsqrt(var + 1e-5) * g_ref[...] + be_ref[...], 0.0)


def _mid_body(acc_ref, t_ref, dinv_ref, b_ref, g_ref, be_ref, w_ref, o_ref):
    h = _postbn(acc_ref, t_ref, dinv_ref, b_ref, g_ref, be_ref)
    o_ref[pl.ds(0, _N), :] = _dot(h, w_ref[...]) * dinv_ref[...]
    o_ref[pl.ds(_N, 8), :] = jnp.zeros((8, _D), jnp.float32)


_mid_call = pl.pallas_call(
    _mid_body, out_shape=jax.ShapeDtypeStruct((_NP, _D), jnp.float32))


def _last_body(acc_ref, t_ref, dinv_ref, b_ref, g_ref, be_ref, o_ref):
    o_ref[...] = _postbn(acc_ref, t_ref, dinv_ref, b_ref, g_ref, be_ref)


_last_call = pl.pallas_call(
    _last_body, out_shape=jax.ShapeDtypeStruct((_N, _D), jnp.float32))


def _cls_body(p_ref, w_ref, b_ref, o_ref):
    o_ref[...] = _dot(p_ref[0] + p_ref[1], w_ref[...]) + b_ref[...]


_cls_call = pl.pallas_call(
    _cls_body, out_shape=jax.ShapeDtypeStruct((_G, _C), jnp.float32))


def _pad_edges(e, fill):
    e2 = e.reshape(_NW, _EPW)
    e2 = jnp.pad(e2, ((0, 0), (0, _EPWP - _EPW)), constant_values=fill)
    return e2.reshape(_NW, _NCHP, _CH)


def kernel(x, edge_index, batch, W0, b0, gamma0, beta0, W1, b1, gamma1,
           beta1, W2, b2, gamma2, beta2, cW, cb):
    src = edge_index[0]
    dst = edge_index[1]
    dst_blk = _pad_edges(edge_index[1], 0)
    zeros_np = jnp.zeros((_N, _D), jnp.float32)
    ones_ch = jnp.ones((_CH, _D), jnp.float32)

    deg_kernel, prop_kernel, pool_kernel = _sc_kernels()

    degacc = deg_kernel(dst_blk, ones_ch, zeros_np)
    xw0 = _first_call(x, W0)
    dinv, t = _dinv_scale_call(degacc, xw0)

    for (b, g, be, Wn) in ((b0, gamma0, beta0, W1), (b1, gamma1, beta1, W2)):
        acc = prop_kernel(t, src, dst, zeros_np)
        t = _mid_call(acc, t, dinv, b.reshape(1, _D), g.reshape(1, _D),
                      be.reshape(1, _D), Wn)
    acc = prop_kernel(t, src, dst, zeros_np)
    h = _last_call(acc, t, dinv, b2.reshape(1, _D), gamma2.reshape(1, _D),
                   beta2.reshape(1, _D))
    pacc = pool_kernel(h, batch, zeros_np)
    return _cls_call(pacc, cW, cb.reshape(1, _C))


# speedup vs baseline: 1.6392x; 1.1627x over previous
import functools

import jax
import jax.numpy as jnp
from jax import lax
from jax.experimental import pallas as pl
from jax.experimental.pallas import tpu as pltpu
from jax.experimental.pallas import tpu_sc as plsc

_N = 10000
_E = 320000
_D = 128
_G = 128
_C = 10

_NC = 2
_NS = 16
_NW = _NC * _NS
_CH = 128
_NCHP = 80
_EPWP = _CH * _NCHP
_EPW = _E // _NW
_NP = _N + 8
_NPAD = _NW * (_EPWP - _EPW)
_RB = 632
_GPS = _G // _NS
_PCH = 80
_PNCH = _EPW // _PCH
_ZB = 80
_NZB = _N // _ZB

_mesh_args = dict(core_axis_name="c", subcore_axis_name="s")


@functools.cache
def _sc_kernels():
    mesh = plsc.VectorSubcoreMesh(**_mesh_args)

    deg = functools.partial(
        pl.kernel,
        out_type=jax.ShapeDtypeStruct((_NC, _N, _D), jnp.float32),
        mesh=mesh,
        scratch_types=[
            pltpu.VMEM((_NCHP, _CH), jnp.int32),
            pltpu.VMEM((_CH, _D), jnp.float32),
            pltpu.VMEM_SHARED((_N, _D), jnp.float32),
        ],
    )(_deg_body)
    prop = functools.partial(
        pl.kernel,
        out_type=jax.ShapeDtypeStruct((_NC, _N, _D), jnp.float32),
        mesh=mesh,
        scratch_types=[
            pltpu.VMEM((_PCH,), jnp.int32),
            pltpu.VMEM((_PCH,), jnp.int32),
            pltpu.VMEM((_PCH,), jnp.int32),
            pltpu.VMEM((_PCH,), jnp.int32),
            pltpu.VMEM((_PCH, _D), jnp.float32),
            pltpu.VMEM((_PCH, _D), jnp.float32),
            pltpu.VMEM_SHARED((_N, _D), jnp.float32),
            pltpu.SemaphoreType.DMA,
            pltpu.SemaphoreType.DMA,
            pltpu.SemaphoreType.DMA,
            pltpu.SemaphoreType.DMA,
        ],
    )(_prop_body)
    pool = functools.partial(
        pl.kernel,
        out_type=jax.ShapeDtypeStruct((_NC, _G, _D), jnp.float32),
        mesh=mesh,
        scratch_types=[
            pltpu.VMEM((_ZB,), jnp.int32),
            pltpu.VMEM((_ZB, _D), jnp.float32),
            pltpu.VMEM_SHARED((_G, _D), jnp.float32),
        ],
    )(_pool_body)
    return deg, prop, pool


def _zero_acc(s, zeros_hbm, acc):
    @pl.when(s < _NS - 1)
    def _():
        off = pl.multiple_of(s * _RB, 8)
        pltpu.sync_copy(zeros_hbm.at[pl.ds(off, _RB)], acc.at[pl.ds(off, _RB)])

    @pl.when(s == _NS - 1)
    def _():
        base = (_NS - 1) * _RB
        pltpu.sync_copy(zeros_hbm.at[pl.ds(base, _N - base)],
                        acc.at[pl.ds(base, _N - base)])


def _writeback(c, s, acc, out_hbm):
    @pl.loop(0, _NZB // _NS + 1)
    def _(j):
        bid = s + j * _NS

        @pl.when(bid < _NZB)
        def _():
            off = pl.multiple_of(bid * _ZB, 8)
            pltpu.sync_copy(acc.at[pl.ds(off, _ZB)],
                            out_hbm.at[c, pl.ds(off, _ZB)])


def _deg_body(dst_hbm, ones_hbm, zeros_hbm, out_hbm, didx, ones_v, acc):
    c = lax.axis_index("c")
    s = lax.axis_index("s")
    wid = c * _NS + s
    _zero_acc(s, zeros_hbm, acc)
    pltpu.sync_copy(dst_hbm.at[wid], didx)
    pltpu.sync_copy(ones_hbm, ones_v)
    plsc.subcore_barrier()

    @pl.loop(0, _NCHP)
    def _(i):
        pltpu.sync_copy(ones_v, acc.at[didx.at[i]], add=True)

    plsc.subcore_barrier()
    _writeback(c, s, acc, out_hbm)


def _prop_body(t_hbm, src_hbm, dst_hbm, zeros_hbm, out_hbm,
               sidx0, didx0, sidx1, didx1, rows0, rows1, acc,
               gsem0, gsem1, ssem0, ssem1):
    c = lax.axis_index("c")
    s = lax.axis_index("s")
    wid = c * _NS + s
    _zero_acc(s, zeros_hbm, acc)
    plsc.subcore_barrier()
    base = wid * _EPW

    @pl.loop(0, (_PNCH - 1) // 2)
    def _(k):
        i = k * 2

        @pl.when(k > 0)
        def _():
            pltpu.make_async_copy(rows0, acc.at[didx0], ssem0).wait()

        off0 = pl.multiple_of(base + i * _PCH, 8)
        pltpu.sync_copy(src_hbm.at[pl.ds(off0, _PCH)], sidx0)
        pltpu.sync_copy(dst_hbm.at[pl.ds(off0, _PCH)], didx0)
        pltpu.async_copy(t_hbm.at[sidx0], rows0, gsem0).wait()
        pltpu.async_copy(rows0, acc.at[didx0], ssem0, add=True)

        @pl.when(k > 0)
        def _():
            pltpu.make_async_copy(rows1, acc.at[didx1], ssem1).wait()

        off1 = pl.multiple_of(base + (i + 1) * _PCH, 8)
        pltpu.sync_copy(src_hbm.at[pl.ds(off1, _PCH)], sidx1)
        pltpu.sync_copy(dst_hbm.at[pl.ds(off1, _PCH)], didx1)
        pltpu.async_copy(t_hbm.at[sidx1], rows1, gsem1).wait()
        pltpu.async_copy(rows1, acc.at[didx1], ssem1, add=True)

    pltpu.make_async_copy(rows0, acc.at[didx0], ssem0).wait()
    pltpu.make_async_copy(rows1, acc.at[didx1], ssem1).wait()
    off = pl.multiple_of(base + (_PNCH - 1) * _PCH, 8)
    pltpu.sync_copy(src_hbm.at[pl.ds(off, _PCH)], sidx0)
    pltpu.sync_copy(dst_hbm.at[pl.ds(off, _PCH)], didx0)
    pltpu.async_copy(t_hbm.at[sidx0], rows0, gsem0).wait()
    pltpu.sync_copy(rows0, acc.at[didx0], add=True)

    plsc.subcore_barrier()
    _writeback(c, s, acc, out_hbm)


def _pool_body(h_hbm, batch_hbm, zeros_hbm, out_hbm, idx_v, rows, acc):
    c = lax.axis_index("c")
    s = lax.axis_index("s")
    wid = c * _NS + s
    pltpu.sync_copy(zeros_hbm.at[pl.ds(s * _GPS, _GPS)],
                    acc.at[pl.ds(s * _GPS, _GPS)])
    plsc.subcore_barrier()

    @pl.loop(0, _NZB // _NW + 1)
    def _(j):
        cid = wid + j * _NW

        @pl.when(cid < _NZB)
        def _():
            off = pl.multiple_of(cid * _ZB, 8)
            pltpu.sync_copy(batch_hbm.at[pl.ds(off, _ZB)], idx_v)
            pltpu.sync_copy(h_hbm.at[pl.ds(off, _ZB)], rows)
            pltpu.sync_copy(rows, acc.at[idx_v], add=True)

    plsc.subcore_barrier()
    pltpu.sync_copy(acc.at[pl.ds(s * _GPS, _GPS)],
                    out_hbm.at[c, pl.ds(s * _GPS, _GPS)])


def _dot(a, b):
    return jnp.dot(a, b, preferred_element_type=jnp.float32,
                   precision=lax.Precision.DEFAULT)


def _first_body(x_ref, w_ref, o_ref):
    o_ref[pl.ds(0, _N), :] = _dot(x_ref[...], w_ref[...])
    o_ref[pl.ds(_N, 8), :] = jnp.zeros((8, _D), jnp.float32)


_first_call = pl.pallas_call(
    _first_body, out_shape=jax.ShapeDtypeStruct((_NP, _D), jnp.float32))


def _dinv_scale_body(degacc_ref, t_ref, dinv_ref, o_ref):
    deg = degacc_ref[0][:, 0:1] + degacc_ref[1][:, 0:1] + 1.0
    row = lax.broadcasted_iota(jnp.int32, (_N, 1), 0)
    deg = deg - jnp.where(row == 0, float(_NPAD), 0.0)
    dinv = lax.rsqrt(jnp.maximum(deg, 1.0))
    dinv_ref[...] = dinv
    o_ref[pl.ds(0, _N), :] = t_ref[pl.ds(0, _N), :] * dinv
    o_ref[pl.ds(_N, 8), :] = jnp.zeros((8, _D), jnp.float32)


_dinv_scale_call = pl.pallas_call(
    _dinv_scale_body,
    out_shape=(jax.ShapeDtypeStruct((_N, 1), jnp.float32),
               jax.ShapeDtypeStruct((_NP, _D), jnp.float32)))


def _postbn(acc_ref, t_ref, dinv_ref, b_ref, g_ref, be_ref):
    t = t_ref[pl.ds(0, _N), :]
    u = (acc_ref[0] + acc_ref[1] + t) * dinv_ref[...] + b_ref[...]
    mean = jnp.mean(u, axis=0, keepdims=True)
    var = jnp.mean((u - mean) ** 2, axis=0, keepdims=True)
    return jnp.maximum(
        (u - mean) * lax.rsqrt(var + 1e-5) * g_ref[...] + be_ref[...], 0.0)


def _mid_body(acc_ref, t_ref, dinv_ref, b_ref, g_ref, be_ref, w_ref, o_ref):
    h = _postbn(acc_ref, t_ref, dinv_ref, b_ref, g_ref, be_ref)
    o_ref[pl.ds(0, _N), :] = _dot(h, w_ref[...]) * dinv_ref[...]
    o_ref[pl.ds(_N, 8), :] = jnp.zeros((8, _D), jnp.float32)


_mid_call = pl.pallas_call(
    _mid_body, out_shape=jax.ShapeDtypeStruct((_NP, _D), jnp.float32))


def _last_body(acc_ref, t_ref, dinv_ref, b_ref, g_ref, be_ref, o_ref):
    o_ref[...] = _postbn(acc_ref, t_ref, dinv_ref, b_ref, g_ref, be_ref)


_last_call = pl.pallas_call(
    _last_body, out_shape=jax.ShapeDtypeStruct((_N, _D), jnp.float32))


def _cls_body(p_ref, w_ref, b_ref, o_ref):
    o_ref[...] = _dot(p_ref[0] + p_ref[1], w_ref[...]) + b_ref[...]


_cls_call = pl.pallas_call(
    _cls_body, out_shape=jax.ShapeDtypeStruct((_G, _C), jnp.float32))


def _pad_edges(e, fill):
    e2 = e.reshape(_NW, _EPW)
    e2 = jnp.pad(e2, ((0, 0), (0, _EPWP - _EPW)), constant_values=fill)
    return e2.reshape(_NW, _NCHP, _CH)


def kernel(x, edge_index, batch, W0, b0, gamma0, beta0, W1, b1, gamma1,
           beta1, W2, b2, gamma2, beta2, cW, cb):
    src = edge_index[0]
    dst = edge_index[1]
    dst_blk = _pad_edges(edge_index[1], 0)
    zeros_np = jnp.zeros((_N, _D), jnp.float32)
    ones_ch = jnp.ones((_CH, _D), jnp.float32)

    deg_kernel, prop_kernel, pool_kernel = _sc_kernels()

    degacc = deg_kernel(dst_blk, ones_ch, zeros_np)
    xw0 = _first_call(x, W0)
    dinv, t = _dinv_scale_call(degacc, xw0)

    for (b, g, be, Wn) in ((b0, gamma0, beta0, W1), (b1, gamma1, beta1, W2)):
        acc = prop_kernel(t, src, dst, zeros_np)
        t = _mid_call(acc, t, dinv, b.reshape(1, _D), g.reshape(1, _D),
                      be.reshape(1, _D), Wn)
    acc = prop_kernel(t, src, dst, zeros_np)
    h = _last_call(acc, t, dinv, b2.reshape(1, _D), gamma2.reshape(1, _D),
                   beta2.reshape(1, _D))
    pacc = pool_kernel(h, batch, zeros_np)
    return _cls_call(pacc, cW, cb.reshape(1, _C))


# speedup vs baseline: 2.0410x; 1.2451x over previous
import functools

import jax
import jax.numpy as jnp
from jax import lax
from jax.experimental import pallas as pl
from jax.experimental.pallas import tpu as pltpu
from jax.experimental.pallas import tpu_sc as plsc

_N = 10000
_E = 320000
_D = 128
_G = 128
_C = 10

_NC = 2
_NS = 16
_NW = _NC * _NS
_CH = 128
_NCHP = 80
_EPWP = _CH * _NCHP
_EPW = _E // _NW
_NP = _N + 8
_NPAD = _NW * (_EPWP - _EPW)
_RB = 632
_GPS = _G // _NS
_PCH = 80
_PNCH = _EPW // _PCH
_ZB = 80
_NZB = _N // _ZB

_mesh_args = dict(core_axis_name="c", subcore_axis_name="s")


@functools.cache
def _sc_kernels():
    mesh = plsc.VectorSubcoreMesh(**_mesh_args)

    deg = functools.partial(
        pl.kernel,
        out_type=jax.ShapeDtypeStruct((_NC, _N, _D), jnp.float32),
        mesh=mesh,
        scratch_types=[
            pltpu.VMEM((_NCHP, _CH), jnp.int32),
            pltpu.VMEM((_CH, _D), jnp.float32),
            pltpu.VMEM_SHARED((_N, _D), jnp.float32),
        ],
    )(_deg_body)
    prop = functools.partial(
        pl.kernel,
        out_type=jax.ShapeDtypeStruct((_NC, _N, _D), jnp.float32),
        mesh=mesh,
        scratch_types=[
            pltpu.VMEM((_PCH,), jnp.int32),
            pltpu.VMEM((_PCH,), jnp.int32),
            pltpu.VMEM((_PCH,), jnp.int32),
            pltpu.VMEM((_PCH,), jnp.int32),
            pltpu.VMEM((_PCH, _D), jnp.float32),
            pltpu.VMEM((_PCH, _D), jnp.float32),
            pltpu.VMEM_SHARED((_N, _D), jnp.float32),
            pltpu.SemaphoreType.DMA,
            pltpu.SemaphoreType.DMA,
            pltpu.SemaphoreType.DMA,
            pltpu.SemaphoreType.DMA,
        ],
    )(_prop_body)
    pool = functools.partial(
        pl.kernel,
        out_type=jax.ShapeDtypeStruct((_NC, _G, _D), jnp.float32),
        mesh=mesh,
        scratch_types=[
            pltpu.VMEM((_ZB,), jnp.int32),
            pltpu.VMEM((_ZB, _D), jnp.float32),
            pltpu.VMEM_SHARED((_G, _D), jnp.float32),
        ],
    )(_pool_body)
    return deg, prop, pool


def _zero_acc(s, zeros_hbm, acc):
    @pl.when(s < _NS - 1)
    def _():
        off = pl.multiple_of(s * _RB, 8)
        pltpu.sync_copy(zeros_hbm.at[pl.ds(off, _RB)], acc.at[pl.ds(off, _RB)])

    @pl.when(s == _NS - 1)
    def _():
        base = (_NS - 1) * _RB
        pltpu.sync_copy(zeros_hbm.at[pl.ds(base, _N - base)],
                        acc.at[pl.ds(base, _N - base)])


def _writeback(c, s, acc, out_hbm):
    @pl.loop(0, _NZB // _NS + 1)
    def _(j):
        bid = s + j * _NS

        @pl.when(bid < _NZB)
        def _():
            off = pl.multiple_of(bid * _ZB, 8)
            pltpu.sync_copy(acc.at[pl.ds(off, _ZB)],
                            out_hbm.at[c, pl.ds(off, _ZB)])


def _deg_body(dst_hbm, ones_hbm, zeros_hbm, out_hbm, didx, ones_v, acc):
    c = lax.axis_index("c")
    s = lax.axis_index("s")
    wid = c * _NS + s
    _zero_acc(s, zeros_hbm, acc)
    pltpu.sync_copy(dst_hbm.at[wid], didx)
    pltpu.sync_copy(ones_hbm, ones_v)
    plsc.subcore_barrier()

    @pl.loop(0, _NCHP)
    def _(i):
        pltpu.sync_copy(ones_v, acc.at[didx.at[i]], add=True)

    plsc.subcore_barrier()
    _writeback(c, s, acc, out_hbm)


def _prop_body(t_hbm, src_hbm, dst_hbm, zeros_hbm, out_hbm,
               sidx0, didx0, sidx1, didx1, rows0, rows1, acc,
               gsem0, gsem1, ssem0, ssem1):
    c = lax.axis_index("c")
    s = lax.axis_index("s")
    wid = c * _NS + s
    _zero_acc(s, zeros_hbm, acc)
    plsc.subcore_barrier()
    base = wid * _EPW

    @pl.loop(0, (_PNCH - 1) // 2)
    def _(k):
        i = k * 2

        @pl.when(k > 0)
        def _():
            pltpu.make_async_copy(rows0, acc.at[didx0], ssem0).wait()

        off0 = pl.multiple_of(base + i * _PCH, 8)
        pltpu.sync_copy(src_hbm.at[pl.ds(off0, _PCH)], sidx0)
        pltpu.sync_copy(dst_hbm.at[pl.ds(off0, _PCH)], didx0)
        pltpu.async_copy(t_hbm.at[sidx0], rows0, gsem0)

        @pl.when(k > 0)
        def _():
            pltpu.make_async_copy(rows1, acc.at[didx1], ssem1).wait()

        off1 = pl.multiple_of(base + (i + 1) * _PCH, 8)
        pltpu.sync_copy(src_hbm.at[pl.ds(off1, _PCH)], sidx1)
        pltpu.sync_copy(dst_hbm.at[pl.ds(off1, _PCH)], didx1)
        pltpu.async_copy(t_hbm.at[sidx1], rows1, gsem1)

        pltpu.make_async_copy(t_hbm.at[sidx0], rows0, gsem0).wait()
        pltpu.async_copy(rows0, acc.at[didx0], ssem0, add=True)
        pltpu.make_async_copy(t_hbm.at[sidx1], rows1, gsem1).wait()
        pltpu.async_copy(rows1, acc.at[didx1], ssem1, add=True)

    pltpu.make_async_copy(rows0, acc.at[didx0], ssem0).wait()
    pltpu.make_async_copy(rows1, acc.at[didx1], ssem1).wait()
    off = pl.multiple_of(base + (_PNCH - 1) * _PCH, 8)
    pltpu.sync_copy(src_hbm.at[pl.ds(off, _PCH)], sidx0)
    pltpu.sync_copy(dst_hbm.at[pl.ds(off, _PCH)], didx0)
    pltpu.async_copy(t_hbm.at[sidx0], rows0, gsem0).wait()
    pltpu.sync_copy(rows0, acc.at[didx0], add=True)

    plsc.subcore_barrier()
    _writeback(c, s, acc, out_hbm)


def _pool_body(h_hbm, batch_hbm, zeros_hbm, out_hbm, idx_v, rows, acc):
    c = lax.axis_index("c")
    s = lax.axis_index("s")
    wid = c * _NS + s
    pltpu.sync_copy(zeros_hbm.at[pl.ds(s * _GPS, _GPS)],
                    acc.at[pl.ds(s * _GPS, _GPS)])
    plsc.subcore_barrier()

    @pl.loop(0, _NZB // _NW + 1)
    def _(j):
        cid = wid + j * _NW

        @pl.when(cid < _NZB)
        def _():
            off = pl.multiple_of(cid * _ZB, 8)
            pltpu.sync_copy(batch_hbm.at[pl.ds(off, _ZB)], idx_v)
            pltpu.sync_copy(h_hbm.at[pl.ds(off, _ZB)], rows)
            pltpu.sync_copy(rows, acc.at[idx_v], add=True)

    plsc.subcore_barrier()
    pltpu.sync_copy(acc.at[pl.ds(s * _GPS, _GPS)],
                    out_hbm.at[c, pl.ds(s * _GPS, _GPS)])


def _dot(a, b):
    return jnp.dot(a, b, preferred_element_type=jnp.float32,
                   precision=lax.Precision.DEFAULT)


def _first_body(x_ref, w_ref, o_ref):
    o_ref[pl.ds(0, _N), :] = _dot(x_ref[...], w_ref[...])
    o_ref[pl.ds(_N, 8), :] = jnp.zeros((8, _D), jnp.float32)


_first_call = pl.pallas_call(
    _first_body, out_shape=jax.ShapeDtypeStruct((_NP, _D), jnp.float32))


def _dinv_scale_body(degacc_ref, t_ref, dinv_ref, o_ref):
    deg = degacc_ref[0][:, 0:1] + degacc_ref[1][:, 0:1] + 1.0
    row = lax.broadcasted_iota(jnp.int32, (_N, 1), 0)
    deg = deg - jnp.where(row == 0, float(_NPAD), 0.0)
    dinv = lax.rsqrt(jnp.maximum(deg, 1.0))
    dinv_ref[...] = dinv
    o_ref[pl.ds(0, _N), :] = t_ref[pl.ds(0, _N), :] * dinv
    o_ref[pl.ds(_N, 8), :] = jnp.zeros((8, _D), jnp.float32)


_dinv_scale_call = pl.pallas_call(
    _dinv_scale_body,
    out_shape=(jax.ShapeDtypeStruct((_N, 1), jnp.float32),
               jax.ShapeDtypeStruct((_NP, _D), jnp.float32)))


def _postbn(acc_ref, t_ref, dinv_ref, b_ref, g_ref, be_ref):
    t = t_ref[pl.ds(0, _N), :]
    u = (acc_ref[0] + acc_ref[1] + t) * dinv_ref[...] + b_ref[...]
    mean = jnp.mean(u, axis=0, keepdims=True)
    var = jnp.mean((u - mean) ** 2, axis=0, keepdims=True)
    return jnp.maximum(
        (u - mean) * lax.rsqrt(var + 1e-5) * g_ref[...] + be_ref[...], 0.0)


def _mid_body(acc_ref, t_ref, dinv_ref, b_ref, g_ref, be_ref, w_ref, o_ref):
    h = _postbn(acc_ref, t_ref, dinv_ref, b_ref, g_ref, be_ref)
    o_ref[pl.ds(0, _N), :] = _dot(h, w_ref[...]) * dinv_ref[...]
    o_ref[pl.ds(_N, 8), :] = jnp.zeros((8, _D), jnp.float32)


_mid_call = pl.pallas_call(
    _mid_body, out_shape=jax.ShapeDtypeStruct((_NP, _D), jnp.float32))


def _last_body(acc_ref, t_ref, dinv_ref, b_ref, g_ref, be_ref, o_ref):
    o_ref[...] = _postbn(acc_ref, t_ref, dinv_ref, b_ref, g_ref, be_ref)


_last_call = pl.pallas_call(
    _last_body, out_shape=jax.ShapeDtypeStruct((_N, _D), jnp.float32))


def _cls_body(p_ref, w_ref, b_ref, o_ref):
    o_ref[...] = _dot(p_ref[0] + p_ref[1], w_ref[...]) + b_ref[...]


_cls_call = pl.pallas_call(
    _cls_body, out_shape=jax.ShapeDtypeStruct((_G, _C), jnp.float32))


def _pad_edges(e, fill):
    e2 = e.reshape(_NW, _EPW)
    e2 = jnp.pad(e2, ((0, 0), (0, _EPWP - _EPW)), constant_values=fill)
    return e2.reshape(_NW, _NCHP, _CH)


def kernel(x, edge_index, batch, W0, b0, gamma0, beta0, W1, b1, gamma1,
           beta1, W2, b2, gamma2, beta2, cW, cb):
    src = edge_index[0]
    dst = edge_index[1]
    dst_blk = _pad_edges(edge_index[1], 0)
    zeros_np = jnp.zeros((_N, _D), jnp.float32)
    ones_ch = jnp.ones((_CH, _D), jnp.float32)

    deg_kernel, prop_kernel, pool_kernel = _sc_kernels()

    degacc = deg_kernel(dst_blk, ones_ch, zeros_np)
    xw0 = _first_call(x, W0)
    dinv, t = _dinv_scale_call(degacc, xw0)

    for (b, g, be, Wn) in ((b0, gamma0, beta0, W1), (b1, gamma1, beta1, W2)):
        acc = prop_kernel(t, src, dst, zeros_np)
        t = _mid_call(acc, t, dinv, b.reshape(1, _D), g.reshape(1, _D),
                      be.reshape(1, _D), Wn)
    acc = prop_kernel(t, src, dst, zeros_np)
    h = _last_call(acc, t, dinv, b2.reshape(1, _D), gamma2.reshape(1, _D),
                   beta2.reshape(1, _D))
    pacc = pool_kernel(h, batch, zeros_np)
    return _cls_call(pacc, cW, cb.reshape(1, _C))


# speedup vs baseline: 2.0447x; 1.0019x over previous
import functools

import jax
import jax.numpy as jnp
from jax import lax
from jax.experimental import pallas as pl
from jax.experimental.pallas import tpu as pltpu
from jax.experimental.pallas import tpu_sc as plsc

_N = 10000
_E = 320000
_D = 128
_G = 128
_C = 10

_NC = 2
_NS = 16
_NW = _NC * _NS
_CH = 128
_NCHP = 80
_EPWP = _CH * _NCHP
_EPW = _E // _NW
_NP = _N + 8
_NPAD = _NW * (_EPWP - _EPW)
_RB = 632
_GPS = _G // _NS
_PCH = 80
_PNCH = _EPW // _PCH
_ZB = 80
_NZB = _N // _ZB

_mesh_args = dict(core_axis_name="c", subcore_axis_name="s")


@functools.cache
def _sc_kernels():
    mesh = plsc.VectorSubcoreMesh(**_mesh_args)

    deg = functools.partial(
        pl.kernel,
        out_type=jax.ShapeDtypeStruct((_NC, _N, _D), jnp.float32),
        mesh=mesh,
        scratch_types=[
            pltpu.VMEM((_NCHP, _CH), jnp.int32),
            pltpu.VMEM((_CH, _D), jnp.float32),
            pltpu.VMEM_SHARED((_N, _D), jnp.float32),
            pltpu.SemaphoreType.DMA,
        ],
    )(_deg_body)
    prop = functools.partial(
        pl.kernel,
        out_type=jax.ShapeDtypeStruct((_NC, _N, _D), jnp.float32),
        mesh=mesh,
        scratch_types=[
            pltpu.VMEM((_PCH,), jnp.int32),
            pltpu.VMEM((_PCH,), jnp.int32),
            pltpu.VMEM((_PCH,), jnp.int32),
            pltpu.VMEM((_PCH,), jnp.int32),
            pltpu.VMEM((_PCH, _D), jnp.float32),
            pltpu.VMEM((_PCH, _D), jnp.float32),
            pltpu.VMEM_SHARED((_N, _D), jnp.float32),
            pltpu.SemaphoreType.DMA,
            pltpu.SemaphoreType.DMA,
            pltpu.SemaphoreType.DMA,
            pltpu.SemaphoreType.DMA,
        ],
    )(_prop_body)
    pool = functools.partial(
        pl.kernel,
        out_type=jax.ShapeDtypeStruct((_NC, _G, _D), jnp.float32),
        mesh=mesh,
        scratch_types=[
            pltpu.VMEM((_ZB,), jnp.int32),
            pltpu.VMEM((_ZB, _D), jnp.float32),
            pltpu.VMEM_SHARED((_G, _D), jnp.float32),
        ],
    )(_pool_body)
    return deg, prop, pool


def _zero_acc(s, zeros_hbm, acc):
    @pl.when(s < _NS - 1)
    def _():
        off = pl.multiple_of(s * _RB, 8)
        pltpu.sync_copy(zeros_hbm.at[pl.ds(off, _RB)], acc.at[pl.ds(off, _RB)])

    @pl.when(s == _NS - 1)
    def _():
        base = (_NS - 1) * _RB
        pltpu.sync_copy(zeros_hbm.at[pl.ds(base, _N - base)],
                        acc.at[pl.ds(base, _N - base)])


def _writeback(c, s, acc, out_hbm):
    @pl.loop(0, _NZB // _NS + 1)
    def _(j):
        bid = s + j * _NS

        @pl.when(bid < _NZB)
        def _():
            off = pl.multiple_of(bid * _ZB, 8)
            pltpu.sync_copy(acc.at[pl.ds(off, _ZB)],
                            out_hbm.at[c, pl.ds(off, _ZB)])


def _deg_body(dst_hbm, ones_hbm, zeros_hbm, out_hbm, didx, ones_v, acc, dsem):
    c = lax.axis_index("c")
    s = lax.axis_index("s")
    wid = c * _NS + s
    _zero_acc(s, zeros_hbm, acc)
    pltpu.sync_copy(dst_hbm.at[wid], didx)
    pltpu.sync_copy(ones_hbm, ones_v)
    plsc.subcore_barrier()

    @pl.loop(0, _NCHP)
    def _(i):
        pltpu.async_copy(ones_v, acc.at[didx.at[i]], dsem, add=True)

    @pl.loop(0, _NCHP)
    def _(i):
        pltpu.make_async_copy(ones_v, acc.at[didx.at[i]], dsem).wait()

    plsc.subcore_barrier()
    _writeback(c, s, acc, out_hbm)


def _prop_body(t_hbm, src_hbm, dst_hbm, zeros_hbm, out_hbm,
               sidx0, didx0, sidx1, didx1, rows0, rows1, acc,
               gsem0, gsem1, ssem0, ssem1):
    c = lax.axis_index("c")
    s = lax.axis_index("s")
    wid = c * _NS + s
    _zero_acc(s, zeros_hbm, acc)
    plsc.subcore_barrier()
    base = wid * _EPW

    @pl.loop(0, (_PNCH - 1) // 2)
    def _(k):
        i = k * 2

        @pl.when(k > 0)
        def _():
            pltpu.make_async_copy(rows0, acc.at[didx0], ssem0).wait()

        off0 = pl.multiple_of(base + i * _PCH, 8)
        pltpu.sync_copy(src_hbm.at[pl.ds(off0, _PCH)], sidx0)
        pltpu.sync_copy(dst_hbm.at[pl.ds(off0, _PCH)], didx0)
        pltpu.async_copy(t_hbm.at[sidx0], rows0, gsem0)

        @pl.when(k > 0)
        def _():
            pltpu.make_async_copy(rows1, acc.at[didx1], ssem1).wait()

        off1 = pl.multiple_of(base + (i + 1) * _PCH, 8)
        pltpu.sync_copy(src_hbm.at[pl.ds(off1, _PCH)], sidx1)
        pltpu.sync_copy(dst_hbm.at[pl.ds(off1, _PCH)], didx1)
        pltpu.async_copy(t_hbm.at[sidx1], rows1, gsem1)

        pltpu.make_async_copy(t_hbm.at[sidx0], rows0, gsem0).wait()
        pltpu.async_copy(rows0, acc.at[didx0], ssem0, add=True)
        pltpu.make_async_copy(t_hbm.at[sidx1], rows1, gsem1).wait()
        pltpu.async_copy(rows1, acc.at[didx1], ssem1, add=True)

    pltpu.make_async_copy(rows0, acc.at[didx0], ssem0).wait()
    pltpu.make_async_copy(rows1, acc.at[didx1], ssem1).wait()
    off = pl.multiple_of(base + (_PNCH - 1) * _PCH, 8)
    pltpu.sync_copy(src_hbm.at[pl.ds(off, _PCH)], sidx0)
    pltpu.sync_copy(dst_hbm.at[pl.ds(off, _PCH)], didx0)
    pltpu.async_copy(t_hbm.at[sidx0], rows0, gsem0).wait()
    pltpu.sync_copy(rows0, acc.at[didx0], add=True)

    plsc.subcore_barrier()
    _writeback(c, s, acc, out_hbm)


def _pool_body(h_hbm, batch_hbm, zeros_hbm, out_hbm, idx_v, rows, acc):
    c = lax.axis_index("c")
    s = lax.axis_index("s")
    wid = c * _NS + s
    pltpu.sync_copy(zeros_hbm.at[pl.ds(s * _GPS, _GPS)],
                    acc.at[pl.ds(s * _GPS, _GPS)])
    plsc.subcore_barrier()

    @pl.loop(0, _NZB // _NW + 1)
    def _(j):
        cid = wid + j * _NW

        @pl.when(cid < _NZB)
        def _():
            off = pl.multiple_of(cid * _ZB, 8)
            pltpu.sync_copy(batch_hbm.at[pl.ds(off, _ZB)], idx_v)
            pltpu.sync_copy(h_hbm.at[pl.ds(off, _ZB)], rows)
            pltpu.sync_copy(rows, acc.at[idx_v], add=True)

    plsc.subcore_barrier()
    pltpu.sync_copy(acc.at[pl.ds(s * _GPS, _GPS)],
                    out_hbm.at[c, pl.ds(s * _GPS, _GPS)])


def _dot(a, b):
    return jnp.dot(a, b, preferred_element_type=jnp.float32,
                   precision=lax.Precision.DEFAULT)


def _first_body(x_ref, w_ref, o_ref):
    o_ref[pl.ds(0, _N), :] = _dot(x_ref[...], w_ref[...])
    o_ref[pl.ds(_N, 8), :] = jnp.zeros((8, _D), jnp.float32)


_first_call = pl.pallas_call(
    _first_body, out_shape=jax.ShapeDtypeStruct((_NP, _D), jnp.float32))


def _dinv_scale_body(degacc_ref, t_ref, dinv_ref, o_ref):
    deg = degacc_ref[0][:, 0:1] + degacc_ref[1][:, 0:1] + 1.0
    row = lax.broadcasted_iota(jnp.int32, (_N, 1), 0)
    deg = deg - jnp.where(row == 0, float(_NPAD), 0.0)
    dinv = lax.rsqrt(jnp.maximum(deg, 1.0))
    dinv_ref[...] = dinv
    o_ref[pl.ds(0, _N), :] = t_ref[pl.ds(0, _N), :] * dinv
    o_ref[pl.ds(_N, 8), :] = jnp.zeros((8, _D), jnp.float32)


_dinv_scale_call = pl.pallas_call(
    _dinv_scale_body,
    out_shape=(jax.ShapeDtypeStruct((_N, 1), jnp.float32),
               jax.ShapeDtypeStruct((_NP, _D), jnp.float32)))


def _postbn(acc_ref, t_ref, dinv_ref, b_ref, g_ref, be_ref):
    t = t_ref[pl.ds(0, _N), :]
    u = (acc_ref[0] + acc_ref[1] + t) * dinv_ref[...] + b_ref[...]
    mean = jnp.mean(u, axis=0, keepdims=True)
    var = jnp.mean((u - mean) ** 2, axis=0, keepdims=True)
    return jnp.maximum(
        (u - mean) * lax.rsqrt(var + 1e-5) * g_ref[...] + be_ref[...], 0.0)


def _mid_body(acc_ref, t_ref, dinv_ref, b_ref, g_ref, be_ref, w_ref, o_ref):
    h = _postbn(acc_ref, t_ref, dinv_ref, b_ref, g_ref, be_ref)
    o_ref[pl.ds(0, _N), :] = _dot(h, w_ref[...]) * dinv_ref[...]
    o_ref[pl.ds(_N, 8), :] = jnp.zeros((8, _D), jnp.float32)


_mid_call = pl.pallas_call(
    _mid_body, out_shape=jax.ShapeDtypeStruct((_NP, _D), jnp.float32))


def _last_body(acc_ref, t_ref, dinv_ref, b_ref, g_ref, be_ref, o_ref):
    o_ref[...] = _postbn(acc_ref, t_ref, dinv_ref, b_ref, g_ref, be_ref)


_last_call = pl.pallas_call(
    _last_body, out_shape=jax.ShapeDtypeStruct((_N, _D), jnp.float32))


def _cls_body(p_ref, w_ref, b_ref, o_ref):
    o_ref[...] = _dot(p_ref[0] + p_ref[1], w_ref[...]) + b_ref[...]


_cls_call = pl.pallas_call(
    _cls_body, out_shape=jax.ShapeDtypeStruct((_G, _C), jnp.float32))


def _pad_edges(e, fill):
    e2 = e.reshape(_NW, _EPW)
    e2 = jnp.pad(e2, ((0, 0), (0, _EPWP - _EPW)), constant_values=fill)
    return e2.reshape(_NW, _NCHP, _CH)


def kernel(x, edge_index, batch, W0, b0, gamma0, beta0, W1, b1, gamma1,
           beta1, W2, b2, gamma2, beta2, cW, cb):
    src = edge_index[0]
    dst = edge_index[1]
    dst_blk = _pad_edges(edge_index[1], 0)
    zeros_np = jnp.zeros((_N, _D), jnp.float32)
    ones_ch = jnp.ones((_CH, _D), jnp.float32)

    deg_kernel, prop_kernel, pool_kernel = _sc_kernels()

    degacc = deg_kernel(dst_blk, ones_ch, zeros_np)
    xw0 = _first_call(x, W0)
    dinv, t = _dinv_scale_call(degacc, xw0)

    for (b, g, be, Wn) in ((b0, gamma0, beta0, W1), (b1, gamma1, beta1, W2)):
        acc = prop_kernel(t, src, dst, zeros_np)
        t = _mid_call(acc, t, dinv, b.reshape(1, _D), g.reshape(1, _D),
                      be.reshape(1, _D), Wn)
    acc = prop_kernel(t, src, dst, zeros_np)
    h = _last_call(acc, t, dinv, b2.reshape(1, _D), gamma2.reshape(1, _D),
                   beta2.reshape(1, _D))
    pacc = pool_kernel(h, batch, zeros_np)
    return _cls_call(pacc, cW, cb.reshape(1, _C))


# speedup vs baseline: 2.3207x; 1.1350x over previous
import functools

import jax
import jax.numpy as jnp
from jax import lax
from jax.experimental import pallas as pl
from jax.experimental.pallas import tpu as pltpu
from jax.experimental.pallas import tpu_sc as plsc

_N = 10000
_E = 320000
_D = 128
_G = 128
_C = 10

_NC = 2
_NS = 16
_NW = _NC * _NS
_CH = 128
_NCHP = 80
_EPWP = _CH * _NCHP
_EPW = _E // _NW
_NP = _N + 8
_NPAD = _NW * (_EPWP - _EPW)
_RB = 632
_GPS = _G // _NS
_PCH = 80
_PNCH = _EPW // _PCH
_ZB = 80
_NZB = _N // _ZB

_mesh_args = dict(core_axis_name="c", subcore_axis_name="s")


@functools.cache
def _sc_kernels():
    mesh = plsc.VectorSubcoreMesh(**_mesh_args)

    deg = functools.partial(
        pl.kernel,
        out_type=jax.ShapeDtypeStruct((_NC, _N, _D), jnp.float32),
        mesh=mesh,
        scratch_types=[
            pltpu.VMEM((_NCHP, _CH), jnp.int32),
            pltpu.VMEM((_CH, _D), jnp.float32),
            pltpu.VMEM_SHARED((_N, _D), jnp.float32),
            pltpu.SemaphoreType.DMA,
        ],
    )(_deg_body)
    prop = functools.partial(
        pl.kernel,
        out_type=jax.ShapeDtypeStruct((_NC, _N, _D), jnp.float32),
        mesh=mesh,
        scratch_types=(
            [pltpu.VMEM((_PCH,), jnp.int32)] * 8
            + [pltpu.VMEM((_PCH, _D), jnp.float32)] * 4
            + [pltpu.VMEM_SHARED((_N, _D), jnp.float32)]
            + [pltpu.SemaphoreType.DMA] * 8
        ),
    )(_prop_body)
    pool = functools.partial(
        pl.kernel,
        out_type=jax.ShapeDtypeStruct((_NC, _G, _D), jnp.float32),
        mesh=mesh,
        scratch_types=[
            pltpu.VMEM((_ZB,), jnp.int32),
            pltpu.VMEM((_ZB, _D), jnp.float32),
            pltpu.VMEM_SHARED((_G, _D), jnp.float32),
        ],
    )(_pool_body)
    return deg, prop, pool


def _zero_acc(s, zeros_hbm, acc):
    @pl.when(s < _NS - 1)
    def _():
        off = pl.multiple_of(s * _RB, 8)
        pltpu.sync_copy(zeros_hbm.at[pl.ds(off, _RB)], acc.at[pl.ds(off, _RB)])

    @pl.when(s == _NS - 1)
    def _():
        base = (_NS - 1) * _RB
        pltpu.sync_copy(zeros_hbm.at[pl.ds(base, _N - base)],
                        acc.at[pl.ds(base, _N - base)])


def _writeback(c, s, acc, out_hbm):
    @pl.loop(0, _NZB // _NS + 1)
    def _(j):
        bid = s + j * _NS

        @pl.when(bid < _NZB)
        def _():
            off = pl.multiple_of(bid * _ZB, 8)
            pltpu.sync_copy(acc.at[pl.ds(off, _ZB)],
                            out_hbm.at[c, pl.ds(off, _ZB)])


def _deg_body(dst_hbm, ones_hbm, zeros_hbm, out_hbm, didx, ones_v, acc, dsem):
    c = lax.axis_index("c")
    s = lax.axis_index("s")
    wid = c * _NS + s
    _zero_acc(s, zeros_hbm, acc)
    pltpu.sync_copy(dst_hbm.at[wid], didx)
    pltpu.sync_copy(ones_hbm, ones_v)
    plsc.subcore_barrier()

    @pl.loop(0, _NCHP)
    def _(i):
        pltpu.async_copy(ones_v, acc.at[didx.at[i]], dsem, add=True)

    @pl.loop(0, _NCHP)
    def _(i):
        pltpu.make_async_copy(ones_v, acc.at[didx.at[i]], dsem).wait()

    plsc.subcore_barrier()
    _writeback(c, s, acc, out_hbm)


_NBUF = 4
_PITER = (_PNCH - 1) // _NBUF


def _prop_body(t_hbm, src_hbm, dst_hbm, zeros_hbm, out_hbm, *scr):
    sidx = scr[0:4]
    didx = scr[4:8]
    rows = scr[8:12]
    acc = scr[12]
    gsem = scr[13:17]
    ssem = scr[17:21]
    c = lax.axis_index("c")
    s = lax.axis_index("s")
    wid = c * _NS + s
    _zero_acc(s, zeros_hbm, acc)
    plsc.subcore_barrier()
    base = wid * _EPW

    @pl.loop(0, _PITER)
    def _(k):
        i = k * _NBUF
        for p in range(_NBUF):
            @pl.when(k > 0)
            def _(p=p):
                pltpu.make_async_copy(rows[p], acc.at[didx[p]],
                                      ssem[p]).wait()

            off = pl.multiple_of(base + (i + p) * _PCH, 8)
            pltpu.sync_copy(src_hbm.at[pl.ds(off, _PCH)], sidx[p])
            pltpu.sync_copy(dst_hbm.at[pl.ds(off, _PCH)], didx[p])
            pltpu.async_copy(t_hbm.at[sidx[p]], rows[p], gsem[p])

        for p in range(_NBUF):
            pltpu.make_async_copy(t_hbm.at[sidx[p]], rows[p], gsem[p]).wait()
            pltpu.async_copy(rows[p], acc.at[didx[p]], ssem[p], add=True)

    for p in range(_NBUF):
        pltpu.make_async_copy(rows[p], acc.at[didx[p]], ssem[p]).wait()
    off = pl.multiple_of(base + _PITER * _NBUF * _PCH, 8)
    pltpu.sync_copy(src_hbm.at[pl.ds(off, _PCH)], sidx[0])
    pltpu.sync_copy(dst_hbm.at[pl.ds(off, _PCH)], didx[0])
    pltpu.async_copy(t_hbm.at[sidx[0]], rows[0], gsem[0]).wait()
    pltpu.sync_copy(rows[0], acc.at[didx[0]], add=True)

    plsc.subcore_barrier()
    _writeback(c, s, acc, out_hbm)


def _pool_body(h_hbm, batch_hbm, zeros_hbm, out_hbm, idx_v, rows, acc):
    c = lax.axis_index("c")
    s = lax.axis_index("s")
    wid = c * _NS + s
    pltpu.sync_copy(zeros_hbm.at[pl.ds(s * _GPS, _GPS)],
                    acc.at[pl.ds(s * _GPS, _GPS)])
    plsc.subcore_barrier()

    @pl.loop(0, _NZB // _NW + 1)
    def _(j):
        cid = wid + j * _NW

        @pl.when(cid < _NZB)
        def _():
            off = pl.multiple_of(cid * _ZB, 8)
            pltpu.sync_copy(batch_hbm.at[pl.ds(off, _ZB)], idx_v)
            pltpu.sync_copy(h_hbm.at[pl.ds(off, _ZB)], rows)
            pltpu.sync_copy(rows, acc.at[idx_v], add=True)

    plsc.subcore_barrier()
    pltpu.sync_copy(acc.at[pl.ds(s * _GPS, _GPS)],
                    out_hbm.at[c, pl.ds(s * _GPS, _GPS)])


def _dot(a, b):
    return jnp.dot(a, b, preferred_element_type=jnp.float32,
                   precision=lax.Precision.DEFAULT)


def _first_body(x_ref, w_ref, o_ref):
    o_ref[pl.ds(0, _N), :] = _dot(x_ref[...], w_ref[...])
    o_ref[pl.ds(_N, 8), :] = jnp.zeros((8, _D), jnp.float32)


_first_call = pl.pallas_call(
    _first_body, out_shape=jax.ShapeDtypeStruct((_NP, _D), jnp.float32))


def _dinv_scale_body(degacc_ref, t_ref, dinv_ref, o_ref):
    deg = degacc_ref[0][:, 0:1] + degacc_ref[1][:, 0:1] + 1.0
    row = lax.broadcasted_iota(jnp.int32, (_N, 1), 0)
    deg = deg - jnp.where(row == 0, float(_NPAD), 0.0)
    dinv = lax.rsqrt(jnp.maximum(deg, 1.0))
    dinv_ref[...] = dinv
    o_ref[pl.ds(0, _N), :] = t_ref[pl.ds(0, _N), :] * dinv
    o_ref[pl.ds(_N, 8), :] = jnp.zeros((8, _D), jnp.float32)


_dinv_scale_call = pl.pallas_call(
    _dinv_scale_body,
    out_shape=(jax.ShapeDtypeStruct((_N, 1), jnp.float32),
               jax.ShapeDtypeStruct((_NP, _D), jnp.float32)))


def _postbn(acc_ref, t_ref, dinv_ref, b_ref, g_ref, be_ref):
    t = t_ref[pl.ds(0, _N), :]
    u = (acc_ref[0] + acc_ref[1] + t) * dinv_ref[...] + b_ref[...]
    mean = jnp.mean(u, axis=0, keepdims=True)
    var = jnp.mean((u - mean) ** 2, axis=0, keepdims=True)
    return jnp.maximum(
        (u - mean) * lax.rsqrt(var + 1e-5) * g_ref[...] + be_ref[...], 0.0)


def _mid_body(acc_ref, t_ref, dinv_ref, b_ref, g_ref, be_ref, w_ref, o_ref):
    h = _postbn(acc_ref, t_ref, dinv_ref, b_ref, g_ref, be_ref)
    o_ref[pl.ds(0, _N), :] = _dot(h, w_ref[...]) * dinv_ref[...]
    o_ref[pl.ds(_N, 8), :] = jnp.zeros((8, _D), jnp.float32)


_mid_call = pl.pallas_call(
    _mid_body, out_shape=jax.ShapeDtypeStruct((_NP, _D), jnp.float32))


def _last_body(acc_ref, t_ref, dinv_ref, b_ref, g_ref, be_ref, o_ref):
    o_ref[...] = _postbn(acc_ref, t_ref, dinv_ref, b_ref, g_ref, be_ref)


_last_call = pl.pallas_call(
    _last_body, out_shape=jax.ShapeDtypeStruct((_N, _D), jnp.float32))


def _cls_body(p_ref, w_ref, b_ref, o_ref):
    o_ref[...] = _dot(p_ref[0] + p_ref[1], w_ref[...]) + b_ref[...]


_cls_call = pl.pallas_call(
    _cls_body, out_shape=jax.ShapeDtypeStruct((_G, _C), jnp.float32))


def _pad_edges(e, fill):
    e2 = e.reshape(_NW, _EPW)
    e2 = jnp.pad(e2, ((0, 0), (0, _EPWP - _EPW)), constant_values=fill)
    return e2.reshape(_NW, _NCHP, _CH)


def kernel(x, edge_index, batch, W0, b0, gamma0, beta0, W1, b1, gamma1,
           beta1, W2, b2, gamma2, beta2, cW, cb):
    src = edge_index[0]
    dst = edge_index[1]
    dst_blk = _pad_edges(edge_index[1], 0)
    zeros_np = jnp.zeros((_N, _D), jnp.float32)
    ones_ch = jnp.ones((_CH, _D), jnp.float32)

    deg_kernel, prop_kernel, pool_kernel = _sc_kernels()

    degacc = deg_kernel(dst_blk, ones_ch, zeros_np)
    xw0 = _first_call(x, W0)
    dinv, t = _dinv_scale_call(degacc, xw0)

    for (b, g, be, Wn) in ((b0, gamma0, beta0, W1), (b1, gamma1, beta1, W2)):
        acc = prop_kernel(t, src, dst, zeros_np)
        t = _mid_call(acc, t, dinv, b.reshape(1, _D), g.reshape(1, _D),
                      be.reshape(1, _D), Wn)
    acc = prop_kernel(t, src, dst, zeros_np)
    h = _last_call(acc, t, dinv, b2.reshape(1, _D), gamma2.reshape(1, _D),
                   beta2.reshape(1, _D))
    pacc = pool_kernel(h, batch, zeros_np)
    return _cls_call(pacc, cW, cb.reshape(1, _C))
